# baseline (speedup 1.0000x reference)
import jax
import jax.numpy as jnp
from jax import lax
from jax.experimental import pallas as pl
from jax.experimental.pallas import tpu as pltpu
from jax.experimental.pallas import tpu_sc as plsc

_NC, _NS, _L = 2, 16, 16
_NW = _NC * _NS

_B, _P, _K = 4096, 2048, 4
_N = _B * _P
_NIN_W = _N // _NW
_CIN = 8192
_NCH = _NIN_W // _CIN
_IT = _CIN // _L


def _sc_body(x_hbm, out_hbm, inbuf, outbuf):
    wid = lax.axis_index("s") * _NC + lax.axis_index("c")
    g = lax.broadcasted_iota(jnp.int32, (_L,), 0)
    gidx = [(g >> 2) + (4 * j) for j in range(4)]
    kvec = ((g & 3) + 1).astype(jnp.float32)
    one = jnp.full((_L,), 1.0, jnp.float32)
    zero = jnp.zeros((_L,), jnp.float32)

    in_base = wid * _NIN_W
    out_base = wid * (_NIN_W * _K)

    def chunk(c, _):
        pltpu.sync_copy(x_hbm.at[pl.ds(in_base + c * _CIN, _CIN)], inbuf)

        def body(i, _):
            ibase = jnp.full((_L,), i * _L, jnp.int32)
            for j in range(4):
                v = plsc.load_gather(inbuf, [ibase + gidx[j]])
                outbuf[pl.ds(i * (4 * _L) + j * _L, _L)] = jnp.where(
                    v == kvec, one, zero)
            return 0

        lax.fori_loop(0, _IT, body, 0)
        pltpu.sync_copy(outbuf,
                        out_hbm.at[pl.ds(out_base + c * _CIN * _K, _CIN * _K)])
        return 0

    lax.fori_loop(0, _NCH, chunk, 0)


_mesh = plsc.VectorSubcoreMesh(core_axis_name="c", subcore_axis_name="s")

_sc_kernel = pl.kernel(
    _sc_body,
    out_type=jax.ShapeDtypeStruct((_N * _K,), jnp.float32),
    mesh=_mesh,
    scratch_types=[
        pltpu.VMEM((_CIN,), jnp.float32),
        pltpu.VMEM((_CIN * _K,), jnp.float32),
    ],
    compiler_params=pltpu.CompilerParams(needs_layout_passes=False),
)


@jax.jit
def kernel(x):
    flat = _sc_kernel(x.reshape(-1))
    return flat.reshape(_B, _P * _K)


# speedup vs baseline: 1.1708x; 1.1708x over previous
import jax
import jax.numpy as jnp
from jax import lax
from jax.experimental import pallas as pl
from jax.experimental.pallas import tpu as pltpu
from jax.experimental.pallas import tpu_sc as plsc

_NC, _NS, _L = 2, 16, 16
_NW = _NC * _NS

_B, _P, _K = 4096, 2048, 4
_N = _B * _P
_NIN_W = _N // _NW
_CIN = 8192
_NCH = _NIN_W // _CIN
_IT = _CIN // _L
_UN = 8


def _sc_body(x_hbm, out_hbm, inb0, inb1, outb0, outb1,
             isem0, isem1, osem0, osem1):
    wid = lax.axis_index("s") * _NC + lax.axis_index("c")
    g = lax.broadcasted_iota(jnp.int32, (_L,), 0)
    gidx = [(g >> 2) + (4 * j) for j in range(4)]
    kvec = ((g & 3) + 1).astype(jnp.float32)
    one = jnp.full((_L,), 1.0, jnp.float32)
    zero = jnp.zeros((_L,), jnp.float32)

    in_base = wid * _NIN_W
    out_base = wid * (_NIN_W * _K)
    inb = [inb0, inb1]
    outb = [outb0, outb1]
    isem = [isem0, isem1]
    osem = [osem0, osem1]

    def in_slice(c):
        return x_hbm.at[pl.ds(in_base + c * _CIN, _CIN)]

    def out_slice(c):
        return out_hbm.at[pl.ds(out_base + c * _CIN * _K, _CIN * _K)]

    def compute(src_ref, dst_ref):
        def body(i, _):
            for u in range(_UN):
                ii = i * _UN + u
                src = src_ref.at[pl.ds(ii * _L, _L)]
                for j in range(4):
                    v = plsc.load_gather(src, [gidx[j]])
                    dst_ref[pl.ds(ii * (4 * _L) + j * _L, _L)] = jnp.where(
                        v == kvec, one, zero)
            return 0
        lax.fori_loop(0, _IT // _UN, body, 0, unroll=False)

    pltpu.async_copy(in_slice(0), inb[0], isem[0])

    def pair(t, _):
        c0 = 2 * t
        pltpu.async_copy(in_slice(c0 + 1), inb[1], isem[1])
        pltpu.make_async_copy(in_slice(c0), inb[0], isem[0]).wait()

        @pl.when(t > 0)
        def _():
            pltpu.make_async_copy(outb[0], out_slice(c0), osem[0]).wait()

        compute(inb[0], outb[0])
        pltpu.async_copy(outb[0], out_slice(c0), osem[0])

        @pl.when(c0 + 2 < _NCH)
        def _():
            pltpu.async_copy(in_slice(c0 + 2), inb[0], isem[0])

        pltpu.make_async_copy(in_slice(c0 + 1), inb[1], isem[1]).wait()

        @pl.when(t > 0)
        def _():
            pltpu.make_async_copy(outb[1], out_slice(c0 + 1), osem[1]).wait()

        compute(inb[1], outb[1])
        pltpu.async_copy(outb[1], out_slice(c0 + 1), osem[1])
        return 0

    lax.fori_loop(0, _NCH // 2, pair, 0, unroll=False)
    pltpu.make_async_copy(outb[0], out_slice(_NCH - 2), osem[0]).wait()
    pltpu.make_async_copy(outb[1], out_slice(_NCH - 1), osem[1]).wait()


_mesh = plsc.VectorSubcoreMesh(core_axis_name="c", subcore_axis_name="s")

_sc_kernel = pl.kernel(
    _sc_body,
    out_type=jax.ShapeDtypeStruct((_N * _K,), jnp.float32),
    mesh=_mesh,
    scratch_types=[
        pltpu.VMEM((_CIN,), jnp.float32),
        pltpu.VMEM((_CIN,), jnp.float32),
        pltpu.VMEM((_CIN * _K,), jnp.float32),
        pltpu.VMEM((_CIN * _K,), jnp.float32),
        pltpu.SemaphoreType.DMA,
        pltpu.SemaphoreType.DMA,
        pltpu.SemaphoreType.DMA,
        pltpu.SemaphoreType.DMA,
    ],
    compiler_params=pltpu.CompilerParams(needs_layout_passes=False),
)


@jax.jit
def kernel(x):
    flat = _sc_kernel(x.reshape(-1))
    return flat.reshape(_B, _P * _K)


# speedup vs baseline: 2.4096x; 2.0580x over previous
import jax
import jax.numpy as jnp
from jax import lax
from jax.experimental import pallas as pl
from jax.experimental.pallas import tpu as pltpu
from jax.experimental.pallas import tpu_sc as plsc

_NC, _NS, _L = 2, 16, 16
_NW = _NC * _NS

_B, _P, _K = 4096, 2048, 4
_N = _B * _P
_NIN_W = _N // _NW
_CIN = 8192
_NCH = _NIN_W // _CIN
_IT = _CIN // _L
_UN = 8


def _sc_body(x_hbm, out_hbm, inb0, inb1, outb0, outb1,
             isem0, isem1, osem0, osem1):
    wid = lax.axis_index("s") * _NC + lax.axis_index("c")
    g = lax.broadcasted_iota(jnp.int32, (_L,), 0)
    sidx = [4 * g + k for k in range(4)]
    kf = [jnp.full((_L,), float(k + 1), jnp.float32) for k in range(4)]
    one = jnp.full((_L,), 1.0, jnp.float32)
    zero = jnp.zeros((_L,), jnp.float32)

    in_base = wid * _NIN_W
    out_base = wid * (_NIN_W * _K)
    inb = [inb0, inb1]
    outb = [outb0, outb1]
    isem = [isem0, isem1]
    osem = [osem0, osem1]

    def in_slice(c):
        return x_hbm.at[pl.ds(in_base + c * _CIN, _CIN)]

    def out_slice(c):
        return out_hbm.at[pl.ds(out_base + c * _CIN * _K, _CIN * _K)]

    def compute(src_ref, dst_ref):
        def body(i, _):
            ii0 = i * _UN
            vs = [src_ref[pl.ds((ii0 + u) * _L, _L)] for u in range(_UN)]
            for u in range(_UN):
                dst64 = dst_ref.at[pl.ds((ii0 + u) * (4 * _L), 4 * _L)]
                for k in range(4):
                    plsc.store_scatter(
                        dst64, [sidx[k]],
                        jnp.where(vs[u] == kf[k], one, zero))
            return 0
        lax.fori_loop(0, _IT // _UN, body, 0, unroll=False)

    pltpu.async_copy(in_slice(0), inb[0], isem[0])

    def pair(t, _):
        c0 = 2 * t
        pltpu.async_copy(in_slice(c0 + 1), inb[1], isem[1])
        pltpu.make_async_copy(in_slice(c0), inb[0], isem[0]).wait()

        @pl.when(t > 0)
        def _():
            pltpu.make_async_copy(outb[0], out_slice(c0), osem[0]).wait()

        compute(inb[0], outb[0])
        pltpu.async_copy(outb[0], out_slice(c0), osem[0])

        @pl.when(c0 + 2 < _NCH)
        def _():
            pltpu.async_copy(in_slice(c0 + 2), inb[0], isem[0])

        pltpu.make_async_copy(in_slice(c0 + 1), inb[1], isem[1]).wait()

        @pl.when(t > 0)
        def _():
            pltpu.make_async_copy(outb[1], out_slice(c0 + 1), osem[1]).wait()

        compute(inb[1], outb[1])
        pltpu.async_copy(outb[1], out_slice(c0 + 1), osem[1])
        return 0

    lax.fori_loop(0, _NCH // 2, pair, 0, unroll=False)
    pltpu.make_async_copy(outb[0], out_slice(_NCH - 2), osem[0]).wait()
    pltpu.make_async_copy(outb[1], out_slice(_NCH - 1), osem[1]).wait()


_mesh = plsc.VectorSubcoreMesh(core_axis_name="c", subcore_axis_name="s")

_sc_kernel = pl.kernel(
    _sc_body,
    out_type=jax.ShapeDtypeStruct((_N * _K,), jnp.float32),
    mesh=_mesh,
    scratch_types=[
        pltpu.VMEM((_CIN,), jnp.float32),
        pltpu.VMEM((_CIN,), jnp.float32),
        pltpu.VMEM((_CIN * _K,), jnp.float32),
        pltpu.VMEM((_CIN * _K,), jnp.float32),
        pltpu.SemaphoreType.DMA,
        pltpu.SemaphoreType.DMA,
        pltpu.SemaphoreType.DMA,
        pltpu.SemaphoreType.DMA,
    ],
    compiler_params=pltpu.CompilerParams(needs_layout_passes=False),
)


@jax.jit
def kernel(x):
    flat = _sc_kernel(x.reshape(-1))
    return flat.reshape(_B, _P * _K)


# speedup vs baseline: 7.4068x; 3.0739x over previous
import jax
import jax.numpy as jnp
from jax import lax
from jax.experimental import pallas as pl
from jax.experimental.pallas import tpu as pltpu
from jax.experimental.pallas import tpu_sc as plsc

_NC, _NS, _L = 2, 16, 16
_NW = _NC * _NS

_B, _P, _K = 4096, 2048, 4
_RW = _B // _NW
_SR = 8
_HC = _P // 2
_NCH = (_RW // _SR) * 2
_UN = 4
_GPR = _HC // _L


def _sc_body(x_hbm, out_hbm, inb0, inb1, outb0, outb1,
             isem0, isem1, osem0, osem1):
    wid = lax.axis_index("s") * _NC + lax.axis_index("c")
    g = lax.broadcasted_iota(jnp.int32, (_L,), 0)
    sidx = [4 * g + k for k in range(4)]
    kf = [jnp.full((_L,), float(k + 1), jnp.float32) for k in range(4)]
    one = jnp.full((_L,), 1.0, jnp.float32)
    zero = jnp.zeros((_L,), jnp.float32)

    row_base = wid * _RW
    inb = [inb0, inb1]
    outb = [outb0, outb1]
    isem = [isem0, isem1]
    osem = [osem0, osem1]

    def start_in(c, buf, sem):
        s, h = c >> 1, c & 1
        for r in range(_SR):
            pltpu.async_copy(
                x_hbm.at[row_base + s * _SR + r, pl.ds(h * _HC, _HC)],
                buf.at[pl.ds(r * _HC, _HC)], sem)

    def wait_in(c, buf, sem):
        s, h = c >> 1, c & 1
        for r in range(_SR):
            pltpu.make_async_copy(
                x_hbm.at[row_base + s * _SR + r, pl.ds(h * _HC, _HC)],
                buf.at[pl.ds(r * _HC, _HC)], sem).wait()

    def start_out(c, buf, sem):
        s, h = c >> 1, c & 1
        for r in range(_SR):
            pltpu.async_copy(
                buf.at[pl.ds(r * _HC * _K, _HC * _K)],
                out_hbm.at[row_base + s * _SR + r,
                           pl.ds(h * _HC * _K, _HC * _K)], sem)

    def wait_out(c, buf, sem):
        s, h = c >> 1, c & 1
        for r in range(_SR):
            pltpu.make_async_copy(
                buf.at[pl.ds(r * _HC * _K, _HC * _K)],
                out_hbm.at[row_base + s * _SR + r,
                           pl.ds(h * _HC * _K, _HC * _K)], sem).wait()

    def compute(src_ref, dst_ref):
        src1 = src_ref
        dst1 = dst_ref

        def body(i, _):
            ii0 = i * _UN
            vs = [src1[pl.ds((ii0 + u) * _L, _L)] for u in range(_UN)]
            for u in range(_UN):
                dst64 = dst1.at[pl.ds((ii0 + u) * (4 * _L), 4 * _L)]
                for k in range(4):
                    plsc.store_scatter(
                        dst64, [sidx[k]],
                        jnp.where(vs[u] == kf[k], one, zero))
            return 0

        lax.fori_loop(0, (_SR * _HC // _L) // _UN, body, 0, unroll=False)

    start_in(0, inb[0], isem[0])

    def pair(t, _):
        c0 = 2 * t
        start_in(c0 + 1, inb[1], isem[1])
        wait_in(c0, inb[0], isem[0])

        @pl.when(t > 0)
        def _():
            wait_out(c0, outb[0], osem[0])

        compute(inb[0], outb[0])
        start_out(c0, outb[0], osem[0])

        @pl.when(c0 + 2 < _NCH)
        def _():
            start_in(c0 + 2, inb[0], isem[0])

        wait_in(c0 + 1, inb[1], isem[1])

        @pl.when(t > 0)
        def _():
            wait_out(c0 + 1, outb[1], osem[1])

        compute(inb[1], outb[1])
        start_out(c0 + 1, outb[1], osem[1])
        return 0

    lax.fori_loop(0, _NCH // 2, pair, 0, unroll=False)
    wait_out(_NCH - 2, outb[0], osem[0])
    wait_out(_NCH - 1, outb[1], osem[1])


_mesh = plsc.VectorSubcoreMesh(core_axis_name="c", subcore_axis_name="s")

_sc_kernel = pl.kernel(
    _sc_body,
    out_type=jax.ShapeDtypeStruct((_B, _P * _K), jnp.float32),
    mesh=_mesh,
    scratch_types=[
        pltpu.VMEM((_SR * _HC,), jnp.float32),
        pltpu.VMEM((_SR * _HC,), jnp.float32),
        pltpu.VMEM((_SR * _HC * _K,), jnp.float32),
        pltpu.VMEM((_SR * _HC * _K,), jnp.float32),
        pltpu.SemaphoreType.DMA,
        pltpu.SemaphoreType.DMA,
        pltpu.SemaphoreType.DMA,
        pltpu.SemaphoreType.DMA,
    ],
    compiler_params=pltpu.CompilerParams(
        needs_layout_passes=False, use_tc_tiling_on_sc=True),
)


@jax.jit
def kernel(x):
    return _sc_kernel(x)
